# Initial kernel scaffold; baseline (speedup 1.0000x reference)
import jax
import jax.numpy as jnp
from jax.experimental import pallas as pl


def kernel(x_sim_m, x_sem_m, x_sim_d, x_sem_d, mm_edge, dd_edge, md_edge, dm_edge, params):
    raise NotImplementedError("write your pallas kernel here")


# speedup vs baseline: 5.1577x; 5.1577x over previous
import functools

import jax
import jax.numpy as jnp
from jax.experimental import pallas as pl
from jax.experimental.pallas import tpu as pltpu

N_M = 10000
N_D = 2048
H = 256
EMB = 256


def _mm_body(x_ref, w_ref, b_ref, o_ref, *, relu, scale_ref=None):
    y = jnp.dot(x_ref[...], w_ref[...], preferred_element_type=jnp.float32)
    y = y + b_ref[...]
    if relu:
        y = jnp.maximum(y, 0.0)
    o_ref[...] = y


def _mm(x, W, b, relu=False, rb=512):
    n, k = x.shape
    m = W.shape[1]
    return pl.pallas_call(
        functools.partial(_mm_body, relu=relu),
        grid=(pl.cdiv(n, rb),),
        in_specs=[
            pl.BlockSpec((rb, k), lambda i: (i, 0)),
            pl.BlockSpec((k, m), lambda i: (0, 0)),
            pl.BlockSpec((1, m), lambda i: (0, 0)),
        ],
        out_specs=pl.BlockSpec((rb, m), lambda i: (i, 0)),
        out_shape=jax.ShapeDtypeStruct((n, m), jnp.float32),
    )(x, W, b.reshape(1, -1))


def _mlp_body(x_ref, w1_ref, b1_ref, w2_ref, b2_ref, o_ref):
    h = jnp.dot(x_ref[...], w1_ref[...], preferred_element_type=jnp.float32)
    h = jnp.maximum(h + b1_ref[...], 0.0)
    y = jnp.dot(h, w2_ref[...], preferred_element_type=jnp.float32)
    o_ref[...] = y + b2_ref[...]


def _mlp(x, p, rb=512):
    n, k = x.shape
    m = p["W2"].shape[1]
    return pl.pallas_call(
        _mlp_body,
        grid=(pl.cdiv(n, rb),),
        in_specs=[
            pl.BlockSpec((rb, k), lambda i: (i, 0)),
            pl.BlockSpec((k, p["W1"].shape[1]), lambda i: (0, 0)),
            pl.BlockSpec((1, p["W1"].shape[1]), lambda i: (0, 0)),
            pl.BlockSpec((p["W1"].shape[1], m), lambda i: (0, 0)),
            pl.BlockSpec((1, m), lambda i: (0, 0)),
        ],
        out_specs=pl.BlockSpec((rb, m), lambda i: (i, 0)),
        out_shape=jax.ShapeDtypeStruct((n, m), jnp.float32),
    )(x, p["W1"], p["b1"].reshape(1, -1), p["W2"], p["b2"].reshape(1, -1))


def _decode_body(a_ref, bq_ref, o_ref):
    o_ref[...] = jax.lax.dot_general(
        a_ref[...], bq_ref[...], (((1,), (1,)), ((), ())),
        preferred_element_type=jnp.float32)


def _decode(zm, zd, P, Q, rb=1024):
    a = _mm(zm, P, jnp.zeros((P.shape[1],), jnp.float32))
    bq = _mm(zd, Q, jnp.zeros((Q.shape[1],), jnp.float32))
    n = a.shape[0]
    return pl.pallas_call(
        _decode_body,
        grid=(pl.cdiv(n, rb),),
        in_specs=[
            pl.BlockSpec((rb, a.shape[1]), lambda i: (i, 0)),
            pl.BlockSpec(bq.shape, lambda i: (0, 0)),
        ],
        out_specs=pl.BlockSpec((rb, bq.shape[0]), lambda i: (i, 0)),
        out_shape=jax.ShapeDtypeStruct((n, bq.shape[0]), jnp.float32),
    )(a, bq)


def _scatter_rows(h, src, dst, n_out):
    return jax.ops.segment_sum(h[src], dst, num_segments=n_out)


def _gcn_pair(x, src, dst, W, b, n, dinv):
    h = _mm(x, W, jnp.zeros_like(b))
    hp = dinv[:, None] * h
    s = _scatter_rows(hp, src, dst, n)
    return dinv[:, None] * (s + hp) + b


def _rgcn(x, rel_edges, Wroot, Wrel, b, dinvs, n):
    out = _mm(x, Wroot, b)
    for r, (src, dst) in enumerate(rel_edges):
        h = _mm(x, Wrel[r], jnp.zeros((Wrel[r].shape[1],), jnp.float32))
        s = _scatter_rows(h, src, dst, n)
        out = out + dinvs[r][:, None] * s
    return out


def kernel(x_sim_m, x_sem_m, x_sim_d, x_sem_d, mm_edge, dd_edge, md_edge, dm_edge, params):
    p = params
    ones_mm = jnp.ones((mm_edge.shape[1],), jnp.float32)
    ones_dd = jnp.ones((dd_edge.shape[1],), jnp.float32)
    ones_md = jnp.ones((md_edge.shape[1],), jnp.float32)
    ones_dm = jnp.ones((dm_edge.shape[1],), jnp.float32)
    cnt_m = jax.ops.segment_sum(ones_mm, mm_edge[1], num_segments=N_M)
    cnt_d = jax.ops.segment_sum(ones_dd, dd_edge[1], num_segments=N_D)
    cnt_md = jax.ops.segment_sum(ones_md, md_edge[1], num_segments=N_D)
    cnt_dm = jax.ops.segment_sum(ones_dm, dm_edge[1], num_segments=N_M)
    dinv_m = jax.lax.rsqrt(cnt_m + 1.0)
    dinv_d = jax.lax.rsqrt(cnt_d + 1.0)
    n_tot = N_M + N_D

    def drecip(c):
        return jnp.where(c > 0, 1.0 / jnp.maximum(c, 1.0), 0.0)

    dinv_r0 = jnp.concatenate([drecip(cnt_m), jnp.zeros((N_D,), jnp.float32)])
    dinv_r1 = jnp.concatenate([jnp.zeros((N_M,), jnp.float32), drecip(cnt_d)])
    dinv_r2 = jnp.concatenate([jnp.zeros((N_M,), jnp.float32), drecip(cnt_md)])
    dinv_r3 = jnp.concatenate([drecip(cnt_dm), jnp.zeros((N_D,), jnp.float32)])

    m_sim = _mlp(x_sim_m, p["m_sim"])
    d_sim = _mlp(x_sim_d, p["d_sim"])
    zm_sim = jnp.maximum(
        _gcn_pair(m_sim, mm_edge[0], mm_edge[1], p["gcn1_W"], p["gcn1_b"], N_M, dinv_m), 0.0)
    zm_sim = _gcn_pair(zm_sim, mm_edge[0], mm_edge[1], p["gcn2_W"], p["gcn2_b"], N_M, dinv_m)
    zd_sim = jnp.maximum(
        _gcn_pair(d_sim, dd_edge[0], dd_edge[1], p["gcn1_W"], p["gcn1_b"], N_D, dinv_d), 0.0)
    zd_sim = _gcn_pair(zd_sim, dd_edge[0], dd_edge[1], p["gcn2_W"], p["gcn2_b"], N_D, dinv_d)

    m_sem = _mlp(x_sem_m, p["m_sem"])
    d_sem = _mlp(x_sem_d, p["d_sem"])
    x = jnp.concatenate([m_sem, d_sem], axis=0)
    rel_edges = [
        (mm_edge[0], mm_edge[1]),
        (dd_edge[0] + N_M, dd_edge[1] + N_M),
        (md_edge[0], md_edge[1] + N_M),
        (dm_edge[0] + N_M, dm_edge[1]),
    ]
    dinvs = [dinv_r0, dinv_r1, dinv_r2, dinv_r3]
    h = _rgcn(x, rel_edges, p["rgcn1_Wroot"], p["rgcn1_Wrel"], p["rgcn1_b"], dinvs, n_tot)
    h = jnp.maximum(h, 0.0)
    h = _rgcn(h, rel_edges, p["rgcn2_Wroot"], p["rgcn2_Wrel"], p["rgcn2_b"], dinvs, n_tot)
    zm_sem = h[:N_M]
    zd_sem = h[N_M:]

    zm = _mm(jnp.concatenate([zm_sim, zm_sem], axis=-1), p["mf_W"], p["mf_b"], relu=True)
    zd = _mm(jnp.concatenate([zd_sim, zd_sem], axis=-1), p["df_W"], p["df_b"], relu=True)
    return _decode(zm, zd, p["P"], p["Q"])


# speedup vs baseline: 12.0433x; 2.3350x over previous
import functools

import jax
import jax.numpy as jnp
from jax import lax
from jax.experimental import pallas as pl
from jax.experimental.pallas import tpu as pltpu
from jax.experimental.pallas import tpu_sc as plsc

N_M = 10000
N_D = 2048
N_MP = 10112
N_DP = 2048
MM_EP = 161792


def _sc_mesh():
    return plsc.VectorSubcoreMesh(core_axis_name="c", subcore_axis_name="s",
                                  num_cores=2, num_subcores=16)


def _zero_zbuf(zbuf, width):
    nchunks = width // 16

    def zb(i, _):
        for j in range(nchunks):
            zbuf[i, pl.ds(j * 16, 16)] = jnp.zeros((16,), jnp.float32)
        return 0

    lax.fori_loop(0, zbuf.shape[0], zb, 0)


def _zero_acc(zbuf, acc, row0, nrows):
    nfull, rem = nrows // 128, nrows % 128

    def za(i, _):
        pltpu.sync_copy(zbuf, acc.at[pl.ds(row0 + i * 128, 128)])
        return 0

    lax.fori_loop(0, nfull, za, 0)
    if rem:
        pltpu.sync_copy(zbuf.at[pl.ds(0, rem)],
                        acc.at[pl.ds(row0 + nfull * 128, rem)])


@functools.partial(jax.jit, static_argnums=(3, 4))
def _sc_scatter(h_flat, srcs2, dst, n_acc, n_edges):
    ept = n_edges // 16
    nit = ept // 128
    ch = n_acc // 16

    @functools.partial(
        pl.kernel,
        out_type=jax.ShapeDtypeStruct((2 * n_acc, 128), jnp.float32),
        mesh=_sc_mesh(),
        scratch_types=[
            pltpu.VMEM((128,), jnp.int32),
            pltpu.VMEM((128,), jnp.int32),
            pltpu.VMEM((128, 128), jnp.float32),
            pltpu.VMEM((128, 128), jnp.float32),
            pltpu.VMEM_SHARED((n_acc, 128), jnp.float32),
            pltpu.SemaphoreType.DMA,
        ],
    )
    def k(h_hbm, srcs_hbm, dst_hbm, out_hbm, idx_s, idx_d, rows, zbuf, acc, sem):
        c = lax.axis_index("c")
        s = lax.axis_index("s")
        _zero_zbuf(zbuf, 128)
        _zero_acc(zbuf, acc, s * ch, ch)
        plsc.subcore_barrier()
        base = s * ept

        def edge_phase(core):
            def body(i, _):
                off = base + i * 128
                pltpu.sync_copy(srcs_hbm.at[core, pl.ds(off, 128)], idx_s)
                pltpu.async_copy(h_hbm.at[idx_s], rows, sem).wait()
                pltpu.sync_copy(dst_hbm.at[pl.ds(off, 128)], idx_d)
                pltpu.sync_copy(rows, acc.at[idx_d], add=True)
                return 0

            lax.fori_loop(0, nit, body, 0)

        @pl.when(c == 0)
        def _():
            edge_phase(0)

        @pl.when(c == 1)
        def _():
            edge_phase(1)

        plsc.subcore_barrier()

        @pl.when(c == 0)
        def _():
            pltpu.sync_copy(acc.at[pl.ds(s * ch, ch)],
                            out_hbm.at[pl.ds(s * ch, ch)])

        @pl.when(c == 1)
        def _():
            pltpu.sync_copy(acc.at[pl.ds(s * ch, ch)],
                            out_hbm.at[pl.ds(n_acc + s * ch, ch)])

    return k(h_flat, srcs2, dst)


_CNT_EM = MM_EP + 65536
_CNT_ED = 32768 + 65536
_CNT_N0 = N_MP + N_DP
_CNT_N1 = N_DP + N_MP


@jax.jit
def _sc_count(dstm, dstd):
    ch0, ch1 = _CNT_N0 // 16, _CNT_N1 // 16
    ept0, ept1 = _CNT_EM // 16, _CNT_ED // 16
    nit0, nit1 = ept0 // 128, ept1 // 128

    @functools.partial(
        pl.kernel,
        out_type=[jax.ShapeDtypeStruct((_CNT_N0, 128), jnp.float32),
                  jax.ShapeDtypeStruct((_CNT_N1, 128), jnp.float32)],
        mesh=_sc_mesh(),
        scratch_types=[
            pltpu.VMEM((128,), jnp.int32),
            pltpu.VMEM((128, 128), jnp.float32),
            pltpu.VMEM((128, 128), jnp.float32),
            pltpu.VMEM_SHARED((_CNT_N0, 128), jnp.float32),
        ],
    )
    def k(dstm_hbm, dstd_hbm, outm, outd, idx_d, ones, zbuf, acc):
        c = lax.axis_index("c")
        s = lax.axis_index("s")

        def ob(i, _):
            for j in range(8):
                ones[i, pl.ds(j * 16, 16)] = jnp.ones((16,), jnp.float32)
            return 0

        lax.fori_loop(0, 128, ob, 0)
        _zero_zbuf(zbuf, 128)
        _zero_acc(zbuf, acc, s * ch0, ch0)
        plsc.subcore_barrier()

        def count_phase(dst_hbm, ept, nit):
            base = s * ept

            def body(i, _):
                pltpu.sync_copy(dst_hbm.at[pl.ds(base + i * 128, 128)], idx_d)
                pltpu.sync_copy(ones, acc.at[idx_d], add=True)
                return 0

            lax.fori_loop(0, nit, body, 0)

        @pl.when(c == 0)
        def _():
            count_phase(dstm_hbm, ept0, nit0)

        @pl.when(c == 1)
        def _():
            count_phase(dstd_hbm, ept1, nit1)

        plsc.subcore_barrier()

        @pl.when(c == 0)
        def _():
            pltpu.sync_copy(acc.at[pl.ds(s * ch0, ch0)],
                            outm.at[pl.ds(s * ch0, ch0)])

        @pl.when(c == 1)
        def _():
            pltpu.sync_copy(acc.at[pl.ds(s * ch1, ch1)],
                            outd.at[pl.ds(s * ch1, ch1)])

    return k(dstm, dstd)


def _mlp_body(x_ref, w1_ref, b1_ref, w2_ref, b2_ref, o_ref):
    h = jnp.dot(x_ref[...], w1_ref[...], preferred_element_type=jnp.float32)
    h = jnp.maximum(h + b1_ref[...], 0.0)
    y = jnp.dot(h, w2_ref[...], preferred_element_type=jnp.float32)
    o_ref[...] = y + b2_ref[...]


def _mlp(x, p, n_out, rb=512):
    k = x.shape[1]
    h1 = p["W1"].shape[1]
    m = p["W2"].shape[1]
    return pl.pallas_call(
        _mlp_body,
        grid=(pl.cdiv(n_out, rb),),
        in_specs=[
            pl.BlockSpec((rb, k), lambda i: (i, 0)),
            pl.BlockSpec((k, h1), lambda i: (0, 0)),
            pl.BlockSpec((1, h1), lambda i: (0, 0)),
            pl.BlockSpec((h1, m), lambda i: (0, 0)),
            pl.BlockSpec((1, m), lambda i: (0, 0)),
        ],
        out_specs=pl.BlockSpec((rb, m), lambda i: (i, 0)),
        out_shape=jax.ShapeDtypeStruct((n_out, m), jnp.float32),
    )(x, p["W1"], p["b1"].reshape(1, -1), p["W2"], p["b2"].reshape(1, -1))


def _mm_parts_body(x_ref, w_ref, cnt_ref, o_ref, *, nparts, scale):
    y = jnp.dot(x_ref[...], w_ref[...], preferred_element_type=jnp.float32)
    if scale:
        y = y * lax.rsqrt(cnt_ref[...][:, 0:1] + 1.0)
    for j in range(nparts):
        o_ref[j] = y[:, j * 128:(j + 1) * 128]


def _mm_parts(x, W, cnt, scale, rb=512):
    n, k = x.shape
    m = W.shape[1]
    nparts = m // 128
    return pl.pallas_call(
        functools.partial(_mm_parts_body, nparts=nparts, scale=scale),
        grid=(pl.cdiv(n, rb),),
        in_specs=[
            pl.BlockSpec((rb, k), lambda i: (i, 0)),
            pl.BlockSpec((k, m), lambda i: (0, 0)),
            pl.BlockSpec((rb, 128), lambda i: (i, 0)),
        ],
        out_specs=pl.BlockSpec((nparts, rb, 128), lambda i: (0, i, 0)),
        out_shape=jax.ShapeDtypeStruct((nparts, n, 128), jnp.float32),
    )(x, W, cnt)


def _cat2(ref):
    return jnp.concatenate([ref[0], ref[1]], axis=-1)


def _recip(cnt_ref):
    c = cnt_ref[...][:, 0:1]
    return jnp.where(c > 0, 1.0 / jnp.maximum(c, 1.0), 0.0)


def _gcn_mid_body(s_ref, hp_ref, cnt_ref, b1_ref, w2_ref, o_ref):
    dinv = lax.rsqrt(cnt_ref[...][:, 0:1] + 1.0)
    z = jnp.maximum(dinv * (_cat2(s_ref) + _cat2(hp_ref)) + b1_ref[...], 0.0)
    y = jnp.dot(z, w2_ref[...], preferred_element_type=jnp.float32) * dinv
    o_ref[0] = y[:, :128]
    o_ref[1] = y[:, 128:]


def _gcn_mid(s1, hp, cnt, b1, W2, rb=512):
    n = hp.shape[1]
    return pl.pallas_call(
        _gcn_mid_body,
        grid=(pl.cdiv(n, rb),),
        in_specs=[
            pl.BlockSpec((2, rb, 128), lambda i: (0, i, 0)),
            pl.BlockSpec((2, rb, 128), lambda i: (0, i, 0)),
            pl.BlockSpec((rb, 128), lambda i: (i, 0)),
            pl.BlockSpec((1, 256), lambda i: (0, 0)),
            pl.BlockSpec((256, 256), lambda i: (0, 0)),
        ],
        out_specs=pl.BlockSpec((2, rb, 128), lambda i: (0, i, 0)),
        out_shape=jax.ShapeDtypeStruct((2, n, 128), jnp.float32),
    )(s1, hp, cnt, b1.reshape(1, -1), W2)


def _gcn_fin_body(s_ref, hp_ref, cnt_ref, b2_ref, o_ref):
    dinv = lax.rsqrt(cnt_ref[...][:, 0:1] + 1.0)
    o_ref[...] = dinv * (_cat2(s_ref) + _cat2(hp_ref)) + b2_ref[...]


def _gcn_fin(s2, hp, cnt, b2, rb=512):
    n = hp.shape[1]
    return pl.pallas_call(
        _gcn_fin_body,
        grid=(pl.cdiv(n, rb),),
        in_specs=[
            pl.BlockSpec((2, rb, 128), lambda i: (0, i, 0)),
            pl.BlockSpec((2, rb, 128), lambda i: (0, i, 0)),
            pl.BlockSpec((rb, 128), lambda i: (i, 0)),
            pl.BlockSpec((1, 256), lambda i: (0, 0)),
        ],
        out_specs=pl.BlockSpec((rb, 256), lambda i: (i, 0)),
        out_shape=jax.ShapeDtypeStruct((n, 256), jnp.float32),
    )(s2, hp, cnt, b2.reshape(1, -1))


def _rgcn_mid_body(h_ref, sa_ref, sb_ref, ca_ref, cb_ref, b_ref, w_ref, o_ref):
    x2 = (_cat2(h_ref) + _recip(ca_ref) * _cat2(sa_ref)
          + _recip(cb_ref) * _cat2(sb_ref) + b_ref[...])
    x2 = jnp.maximum(x2, 0.0)
    y = jnp.dot(x2, w_ref[...], preferred_element_type=jnp.float32)
    for j in range(6):
        o_ref[j] = y[:, j * 128:(j + 1) * 128]


def _rgcn_mid(H, sa, sb, ca, cb, b1, Wcat2, rb=512):
    n = H.shape[1]
    return pl.pallas_call(
        _rgcn_mid_body,
        grid=(pl.cdiv(n, rb),),
        in_specs=[
            pl.BlockSpec((2, rb, 128), lambda i: (0, i, 0)),
            pl.BlockSpec((2, rb, 128), lambda i: (0, i, 0)),
            pl.BlockSpec((2, rb, 128), lambda i: (0, i, 0)),
            pl.BlockSpec((rb, 128), lambda i: (i, 0)),
            pl.BlockSpec((rb, 128), lambda i: (i, 0)),
            pl.BlockSpec((1, 256), lambda i: (0, 0)),
            pl.BlockSpec((256, 768), lambda i: (0, 0)),
        ],
        out_specs=pl.BlockSpec((6, rb, 128), lambda i: (0, i, 0)),
        out_shape=jax.ShapeDtypeStruct((6, n, 128), jnp.float32),
    )(H, sa, sb, ca, cb, b1.reshape(1, -1), Wcat2)


def _rgcn_fin_body(h_ref, sa_ref, sb_ref, ca_ref, cb_ref, b_ref, o_ref):
    o_ref[...] = (_cat2(h_ref) + _recip(ca_ref) * _cat2(sa_ref)
                  + _recip(cb_ref) * _cat2(sb_ref) + b_ref[...])


def _rgcn_fin(H, sa, sb, ca, cb, b2, rb=512):
    n = H.shape[1]
    return pl.pallas_call(
        _rgcn_fin_body,
        grid=(pl.cdiv(n, rb),),
        in_specs=[
            pl.BlockSpec((2, rb, 128), lambda i: (0, i, 0)),
            pl.BlockSpec((2, rb, 128), lambda i: (0, i, 0)),
            pl.BlockSpec((2, rb, 128), lambda i: (0, i, 0)),
            pl.BlockSpec((rb, 128), lambda i: (i, 0)),
            pl.BlockSpec((rb, 128), lambda i: (i, 0)),
            pl.BlockSpec((1, 256), lambda i: (0, 0)),
        ],
        out_specs=pl.BlockSpec((rb, 256), lambda i: (i, 0)),
        out_shape=jax.ShapeDtypeStruct((n, 256), jnp.float32),
    )(H, sa, sb, ca, cb, b2.reshape(1, -1))


def _fuse2_body(a_ref, b_ref, wt_ref, wb_ref, bias_ref, o_ref):
    y = jnp.dot(a_ref[...], wt_ref[...], preferred_element_type=jnp.float32)
    y = y + jnp.dot(b_ref[...], wb_ref[...], preferred_element_type=jnp.float32)
    o_ref[...] = jnp.maximum(y + bias_ref[...], 0.0)


def _fuse2(za, zb, W, b, rb=512):
    n = za.shape[0]
    m = W.shape[1]
    return pl.pallas_call(
        _fuse2_body,
        grid=(pl.cdiv(n, rb),),
        in_specs=[
            pl.BlockSpec((rb, 256), lambda i: (i, 0)),
            pl.BlockSpec((rb, 256), lambda i: (i, 0)),
            pl.BlockSpec((256, m), lambda i: (0, 0)),
            pl.BlockSpec((256, m), lambda i: (0, 0)),
            pl.BlockSpec((1, m), lambda i: (0, 0)),
        ],
        out_specs=pl.BlockSpec((rb, m), lambda i: (i, 0)),
        out_shape=jax.ShapeDtypeStruct((n, m), jnp.float32),
    )(za, zb, W[:256], W[256:], b.reshape(1, -1))


def _mm_body(x_ref, w_ref, o_ref):
    o_ref[...] = jnp.dot(x_ref[...], w_ref[...],
                         preferred_element_type=jnp.float32)


def _mm(x, W, rb=1024):
    n, k = x.shape
    m = W.shape[1]
    return pl.pallas_call(
        _mm_body,
        grid=(pl.cdiv(n, rb),),
        in_specs=[
            pl.BlockSpec((rb, k), lambda i: (i, 0)),
            pl.BlockSpec((k, m), lambda i: (0, 0)),
        ],
        out_specs=pl.BlockSpec((rb, m), lambda i: (i, 0)),
        out_shape=jax.ShapeDtypeStruct((n, m), jnp.float32),
    )(x, W)


def _decode_body(a_ref, bq_ref, o_ref):
    o_ref[...] = lax.dot_general(
        a_ref[...], bq_ref[...], (((1,), (1,)), ((), ())),
        preferred_element_type=jnp.float32)


def _decode(zm, zd, P, Q, rb=1024):
    a = _mm(zm, P)
    bq = _mm(zd, Q)
    n = a.shape[0]
    return pl.pallas_call(
        _decode_body,
        grid=(pl.cdiv(n, rb),),
        in_specs=[
            pl.BlockSpec((rb, a.shape[1]), lambda i: (i, 0)),
            pl.BlockSpec(bq.shape, lambda i: (0, 0)),
        ],
        out_specs=pl.BlockSpec((rb, bq.shape[0]), lambda i: (i, 0)),
        out_shape=jax.ShapeDtypeStruct((n, bq.shape[0]), jnp.float32),
    )(a, bq)


def _srcs2(src, base_part, n_rows):
    return jnp.stack([src + base_part * n_rows, src + (base_part + 1) * n_rows])


def _gcn2(x, src, dst, cnt, p, n, n_edges):
    hp = _mm_parts(x, p["gcn1_W"], cnt, scale=True)
    s1 = _sc_scatter(hp.reshape(-1, 128), _srcs2(src, 0, n), dst, n, n_edges)
    h2p = _gcn_mid(s1.reshape(2, n, 128), hp, cnt, p["gcn1_b"], p["gcn2_W"])
    s2 = _sc_scatter(h2p.reshape(-1, 128), _srcs2(src, 0, n), dst, n, n_edges)
    return _gcn_fin(s2.reshape(2, n, 128), h2p, cnt, p["gcn2_b"])


def kernel(x_sim_m, x_sem_m, x_sim_d, x_sem_d, mm_edge, dd_edge, md_edge, dm_edge, params):
    p = params
    i32 = jnp.int32
    npad = MM_EP - mm_edge.shape[1]
    mm_src = jnp.concatenate([mm_edge[0], jnp.zeros((npad,), i32)])
    mm_dst = jnp.concatenate([mm_edge[1], jnp.full((npad,), N_M + 8, i32)])

    dstm_list = jnp.concatenate([mm_dst, md_edge[1] + N_MP])
    dstd_list = jnp.concatenate([dd_edge[1], dm_edge[1] + N_DP])
    cnt0, cnt1 = _sc_count(dstm_list, dstd_list)
    cnt_mm, cnt_md = cnt0[:N_MP], cnt0[N_MP:]
    cnt_dd, cnt_dm = cnt1[:N_DP], cnt1[N_DP:]

    m_sim = _mlp(x_sim_m, p["m_sim"], N_MP)
    d_sim = _mlp(x_sim_d, p["d_sim"], N_DP)
    zm_sim = _gcn2(m_sim, mm_src, mm_dst, cnt_mm, p, N_MP, MM_EP)
    zd_sim = _gcn2(d_sim, dd_edge[0], dd_edge[1], cnt_dd, p, N_DP, 32768)

    m_sem = _mlp(x_sem_m, p["m_sem"], N_MP)
    d_sem = _mlp(x_sem_d, p["d_sem"], N_DP)
    Wm1 = jnp.concatenate([p["rgcn1_Wroot"], p["rgcn1_Wrel"][0], p["rgcn1_Wrel"][2]], axis=1)
    Wd1 = jnp.concatenate([p["rgcn1_Wroot"], p["rgcn1_Wrel"][1], p["rgcn1_Wrel"][3]], axis=1)
    Wm2 = jnp.concatenate([p["rgcn2_Wroot"], p["rgcn2_Wrel"][0], p["rgcn2_Wrel"][2]], axis=1)
    Wd2 = jnp.concatenate([p["rgcn2_Wroot"], p["rgcn2_Wrel"][1], p["rgcn2_Wrel"][3]], axis=1)
    dummy_cnt_m = cnt_mm
    Hm = _mm_parts(m_sem, Wm1, dummy_cnt_m, scale=False)
    Hd = _mm_parts(d_sem, Wd1, cnt_dd, scale=False)

    def rel_scatters(Hm_, Hd_):
        hmf = Hm_.reshape(-1, 128)
        hdf = Hd_.reshape(-1, 128)
        s0 = _sc_scatter(hmf, _srcs2(mm_src, 2, N_MP), mm_dst, N_MP, MM_EP)
        s2 = _sc_scatter(hmf, _srcs2(md_edge[0], 4, N_MP), md_edge[1], N_DP, 65536)
        s1 = _sc_scatter(hdf, _srcs2(dd_edge[0], 2, N_DP), dd_edge[1], N_DP, 32768)
        s3 = _sc_scatter(hdf, _srcs2(dm_edge[0], 4, N_DP), dm_edge[1], N_MP, 65536)
        return (s0.reshape(2, N_MP, 128), s1.reshape(2, N_DP, 128),
                s2.reshape(2, N_DP, 128), s3.reshape(2, N_MP, 128))

    s0, s1, s2, s3 = rel_scatters(Hm, Hd)
    Hm = _rgcn_mid(Hm, s0, s3, cnt_mm, cnt_dm, p["rgcn1_b"], Wm2)
    Hd = _rgcn_mid(Hd, s1, s2, cnt_dd, cnt_md, p["rgcn1_b"], Wd2)
    s0, s1, s2, s3 = rel_scatters(Hm, Hd)
    zm_sem = _rgcn_fin(Hm, s0, s3, cnt_mm, cnt_dm, p["rgcn2_b"])
    zd_sem = _rgcn_fin(Hd, s1, s2, cnt_dd, cnt_md, p["rgcn2_b"])

    zm = _fuse2(zm_sim, zm_sem, p["mf_W"], p["mf_b"])
    zd = _fuse2(zd_sim, zd_sem, p["df_W"], p["df_b"])
    logits = _decode(zm, zd, p["P"], p["Q"])
    return logits[:N_M, :N_D]


# speedup vs baseline: 14.0176x; 1.1639x over previous
import functools

import jax
import jax.numpy as jnp
from jax import lax
from jax.experimental import pallas as pl
from jax.experimental.pallas import tpu as pltpu
from jax.experimental.pallas import tpu_sc as plsc

N_M = 10000
N_D = 2048
N_MP = 10112
N_DP = 2048
MM_EP = 163840
NBUF = 4


def _sc_mesh():
    return plsc.VectorSubcoreMesh(core_axis_name="c", subcore_axis_name="s",
                                  num_cores=2, num_subcores=16)


def _zero_zbuf(zbuf, width):
    nchunks = width // 16

    def zb(i, _):
        for j in range(nchunks):
            zbuf[i, pl.ds(j * 16, 16)] = jnp.zeros((16,), jnp.float32)
        return 0

    lax.fori_loop(0, zbuf.shape[0], zb, 0)


def _zero_acc(zbuf, acc, row0, nrows):
    nfull, rem = nrows // 128, nrows % 128

    def za(i, _):
        pltpu.sync_copy(zbuf, acc.at[pl.ds(row0 + i * 128, 128)])
        return 0

    lax.fori_loop(0, nfull, za, 0)
    if rem:
        pltpu.sync_copy(zbuf.at[pl.ds(0, rem)],
                        acc.at[pl.ds(row0 + nfull * 128, rem)])


@functools.partial(jax.jit, static_argnums=(3, 4))
def _sc_scatter(h_flat, srcs3, dst3, n_acc, n_edges):
    ept = n_edges // 16
    nit = ept // 128
    ch = n_acc // 16

    @functools.partial(
        pl.kernel,
        out_type=jax.ShapeDtypeStruct((2 * n_acc, 128), jnp.float32),
        mesh=_sc_mesh(),
        scratch_types=[
            pltpu.VMEM((2, 128), jnp.int32),
            pltpu.VMEM((2, 128), jnp.int32),
            pltpu.VMEM((2, 128, 128), jnp.float32),
            pltpu.VMEM_SHARED((n_acc, 128), jnp.float32),
        ] + [pltpu.SemaphoreType.DMA] * 6,
    )
    def k(h_hbm, srcs_hbm, dst_hbm, out_hbm, sidx, didx, rows, acc,
          is0, is1, js0, js1, gs0, gs1):
        c = lax.axis_index("c")
        s = lax.axis_index("s")
        isems = (is0, is1)
        jsems = (js0, js1)
        gsems = (gs0, gs1)
        zbuf = rows.at[0]
        _zero_zbuf(zbuf, 128)
        _zero_acc(zbuf, acc, s * ch, ch)
        plsc.subcore_barrier()
        base = s * ept

        def run(core):
            def istart(i, b):
                off = base + i * 128
                pltpu.async_copy(srcs_hbm.at[core, pl.ds(off, 128)],
                                 sidx.at[b], isems[b])
                pltpu.async_copy(dst_hbm.at[pl.ds(off, 128)],
                                 didx.at[b], jsems[b])

            def iwait(i, b):
                off = base + i * 128
                pltpu.make_async_copy(srcs_hbm.at[core, pl.ds(off, 128)],
                                      sidx.at[b], isems[b]).wait()
                pltpu.make_async_copy(dst_hbm.at[pl.ds(off, 128)],
                                      didx.at[b], jsems[b]).wait()

            def gstart(b):
                pltpu.async_copy(h_hbm.at[sidx.at[b]], rows.at[b], gsems[b])

            def gwait(b):
                pltpu.make_async_copy(h_hbm.at[sidx.at[b]], rows.at[b],
                                      gsems[b]).wait()

            istart(0, 0)
            istart(1, 1)
            iwait(0, 0)
            gstart(0)

            def body(g, _):
                for b in range(2):
                    i = 2 * g + b

                    @pl.when(i + 1 < nit)
                    def _():
                        iwait(i + 1, 1 - b)
                        gstart(1 - b)

                    gwait(b)
                    pltpu.sync_copy(rows.at[b], acc.at[didx.at[b]], add=True)

                    @pl.when(i + 2 < nit)
                    def _():
                        istart(i + 2, b)

                return 0

            lax.fori_loop(0, nit // 2, body, 0)

        @pl.when(c == 0)
        def _():
            run(0)

        @pl.when(c == 1)
        def _():
            run(1)

        plsc.subcore_barrier()

        @pl.when(c == 0)
        def _():
            pltpu.sync_copy(acc.at[pl.ds(s * ch, ch)],
                            out_hbm.at[pl.ds(s * ch, ch)])

        @pl.when(c == 1)
        def _():
            pltpu.sync_copy(acc.at[pl.ds(s * ch, ch)],
                            out_hbm.at[pl.ds(n_acc + s * ch, ch)])

    return k(h_flat, srcs3, dst3)


_CNT_EM = MM_EP + 65536
_CNT_ED = 32768 + 65536
_CNT_N0 = N_MP + N_DP
_CNT_N1 = N_DP + N_MP


@jax.jit
def _sc_count(dstm, dstd):
    ch0, ch1 = _CNT_N0 // 16, _CNT_N1 // 16
    ept0, ept1 = _CNT_EM // 16, _CNT_ED // 16
    nit0, nit1 = ept0 // 128, ept1 // 128

    @functools.partial(
        pl.kernel,
        out_type=[jax.ShapeDtypeStruct((_CNT_N0, 128), jnp.float32),
                  jax.ShapeDtypeStruct((_CNT_N1, 128), jnp.float32)],
        mesh=_sc_mesh(),
        scratch_types=[
            pltpu.VMEM((128,), jnp.int32),
            pltpu.VMEM((128, 128), jnp.float32),
            pltpu.VMEM((128, 128), jnp.float32),
            pltpu.VMEM_SHARED((_CNT_N0, 128), jnp.float32),
        ],
    )
    def k(dstm_hbm, dstd_hbm, outm, outd, idx_d, ones, zbuf, acc):
        c = lax.axis_index("c")
        s = lax.axis_index("s")

        def ob(i, _):
            for j in range(8):
                ones[i, pl.ds(j * 16, 16)] = jnp.ones((16,), jnp.float32)
            return 0

        lax.fori_loop(0, 128, ob, 0)
        _zero_zbuf(zbuf, 128)
        _zero_acc(zbuf, acc, s * ch0, ch0)
        plsc.subcore_barrier()

        def count_phase(dst_hbm, nit):
            base = s * nit * 128

            def body(i, _):
                pltpu.sync_copy(dst_hbm.at[pl.ds(base + i * 128, 128)], idx_d)
                pltpu.sync_copy(ones, acc.at[idx_d], add=True)
                return 0

            lax.fori_loop(0, nit, body, 0)

        @pl.when(c == 0)
        def _():
            count_phase(dstm_hbm, nit0)

        @pl.when(c == 1)
        def _():
            count_phase(dstd_hbm, nit1)

        plsc.subcore_barrier()

        @pl.when(c == 0)
        def _():
            pltpu.sync_copy(acc.at[pl.ds(s * ch0, ch0)],
                            outm.at[pl.ds(s * ch0, ch0)])

        @pl.when(c == 1)
        def _():
            pltpu.sync_copy(acc.at[pl.ds(s * ch1, ch1)],
                            outd.at[pl.ds(s * ch1, ch1)])

    return k(dstm, dstd)


def _mlp_body(x_ref, w1_ref, b1_ref, w2_ref, b2_ref, o_ref):
    h = jnp.dot(x_ref[...], w1_ref[...], preferred_element_type=jnp.float32)
    h = jnp.maximum(h + b1_ref[...], 0.0)
    y = jnp.dot(h, w2_ref[...], preferred_element_type=jnp.float32)
    o_ref[...] = y + b2_ref[...]


def _mlp(x, p, n_out, rb=512):
    k = x.shape[1]
    h1 = p["W1"].shape[1]
    m = p["W2"].shape[1]
    return pl.pallas_call(
        _mlp_body,
        grid=(pl.cdiv(n_out, rb),),
        in_specs=[
            pl.BlockSpec((rb, k), lambda i: (i, 0)),
            pl.BlockSpec((k, h1), lambda i: (0, 0)),
            pl.BlockSpec((1, h1), lambda i: (0, 0)),
            pl.BlockSpec((h1, m), lambda i: (0, 0)),
            pl.BlockSpec((1, m), lambda i: (0, 0)),
        ],
        out_specs=pl.BlockSpec((rb, m), lambda i: (i, 0)),
        out_shape=jax.ShapeDtypeStruct((n_out, m), jnp.float32),
    )(x, p["W1"], p["b1"].reshape(1, -1), p["W2"], p["b2"].reshape(1, -1))


def _mm_parts_body(x_ref, w_ref, cnt_ref, o_ref, *, nparts, scale):
    y = jnp.dot(x_ref[...], w_ref[...], preferred_element_type=jnp.float32)
    if scale:
        y = y * lax.rsqrt(cnt_ref[...][:, 0:1] + 1.0)
    for j in range(nparts):
        o_ref[j] = y[:, j * 128:(j + 1) * 128]


def _mm_parts(x, W, cnt, scale, rb=512):
    n, k = x.shape
    m = W.shape[1]
    nparts = m // 128
    return pl.pallas_call(
        functools.partial(_mm_parts_body, nparts=nparts, scale=scale),
        grid=(pl.cdiv(n, rb),),
        in_specs=[
            pl.BlockSpec((rb, k), lambda i: (i, 0)),
            pl.BlockSpec((k, m), lambda i: (0, 0)),
            pl.BlockSpec((rb, 128), lambda i: (i, 0)),
        ],
        out_specs=pl.BlockSpec((nparts, rb, 128), lambda i: (0, i, 0)),
        out_shape=jax.ShapeDtypeStruct((nparts, n, 128), jnp.float32),
    )(x, W, cnt)


def _cat2(ref):
    return jnp.concatenate([ref[0], ref[1]], axis=-1)


def _recip(cnt_ref):
    c = cnt_ref[...][:, 0:1]
    return jnp.where(c > 0, 1.0 / jnp.maximum(c, 1.0), 0.0)


def _gcn_mid_body(s_ref, hp_ref, cnt_ref, b1_ref, w2_ref, o_ref):
    dinv = lax.rsqrt(cnt_ref[...][:, 0:1] + 1.0)
    z = jnp.maximum(dinv * (_cat2(s_ref) + _cat2(hp_ref)) + b1_ref[...], 0.0)
    y = jnp.dot(z, w2_ref[...], preferred_element_type=jnp.float32) * dinv
    o_ref[0] = y[:, :128]
    o_ref[1] = y[:, 128:]


def _gcn_mid(s1, hp, cnt, b1, W2, rb=512):
    n = hp.shape[1]
    return pl.pallas_call(
        _gcn_mid_body,
        grid=(pl.cdiv(n, rb),),
        in_specs=[
            pl.BlockSpec((2, rb, 128), lambda i: (0, i, 0)),
            pl.BlockSpec((2, rb, 128), lambda i: (0, i, 0)),
            pl.BlockSpec((rb, 128), lambda i: (i, 0)),
            pl.BlockSpec((1, 256), lambda i: (0, 0)),
            pl.BlockSpec((256, 256), lambda i: (0, 0)),
        ],
        out_specs=pl.BlockSpec((2, rb, 128), lambda i: (0, i, 0)),
        out_shape=jax.ShapeDtypeStruct((2, n, 128), jnp.float32),
    )(s1, hp, cnt, b1.reshape(1, -1), W2)


def _gcn_fin_body(s_ref, hp_ref, cnt_ref, b2_ref, o_ref):
    dinv = lax.rsqrt(cnt_ref[...][:, 0:1] + 1.0)
    o_ref[...] = dinv * (_cat2(s_ref) + _cat2(hp_ref)) + b2_ref[...]


def _gcn_fin(s2, hp, cnt, b2, rb=512):
    n = hp.shape[1]
    return pl.pallas_call(
        _gcn_fin_body,
        grid=(pl.cdiv(n, rb),),
        in_specs=[
            pl.BlockSpec((2, rb, 128), lambda i: (0, i, 0)),
            pl.BlockSpec((2, rb, 128), lambda i: (0, i, 0)),
            pl.BlockSpec((rb, 128), lambda i: (i, 0)),
            pl.BlockSpec((1, 256), lambda i: (0, 0)),
        ],
        out_specs=pl.BlockSpec((rb, 256), lambda i: (i, 0)),
        out_shape=jax.ShapeDtypeStruct((n, 256), jnp.float32),
    )(s2, hp, cnt, b2.reshape(1, -1))


def _rgcn_mid_body(h_ref, sa_ref, sb_ref, ca_ref, cb_ref, b_ref, w_ref, o_ref):
    x2 = (_cat2(h_ref) + _recip(ca_ref) * _cat2(sa_ref)
          + _recip(cb_ref) * _cat2(sb_ref) + b_ref[...])
    x2 = jnp.maximum(x2, 0.0)
    y = jnp.dot(x2, w_ref[...], preferred_element_type=jnp.float32)
    for j in range(6):
        o_ref[j] = y[:, j * 128:(j + 1) * 128]


def _rgcn_mid(H, sa, sb, ca, cb, b1, Wcat2, rb=512):
    n = H.shape[1]
    return pl.pallas_call(
        _rgcn_mid_body,
        grid=(pl.cdiv(n, rb),),
        in_specs=[
            pl.BlockSpec((2, rb, 128), lambda i: (0, i, 0)),
            pl.BlockSpec((2, rb, 128), lambda i: (0, i, 0)),
            pl.BlockSpec((2, rb, 128), lambda i: (0, i, 0)),
            pl.BlockSpec((rb, 128), lambda i: (i, 0)),
            pl.BlockSpec((rb, 128), lambda i: (i, 0)),
            pl.BlockSpec((1, 256), lambda i: (0, 0)),
            pl.BlockSpec((256, 768), lambda i: (0, 0)),
        ],
        out_specs=pl.BlockSpec((6, rb, 128), lambda i: (0, i, 0)),
        out_shape=jax.ShapeDtypeStruct((6, n, 128), jnp.float32),
    )(H, sa, sb, ca, cb, b1.reshape(1, -1), Wcat2)


def _rgcn_fin_body(h_ref, sa_ref, sb_ref, ca_ref, cb_ref, b_ref, o_ref):
    o_ref[...] = (_cat2(h_ref) + _recip(ca_ref) * _cat2(sa_ref)
                  + _recip(cb_ref) * _cat2(sb_ref) + b_ref[...])


def _rgcn_fin(H, sa, sb, ca, cb, b2, rb=512):
    n = H.shape[1]
    return pl.pallas_call(
        _rgcn_fin_body,
        grid=(pl.cdiv(n, rb),),
        in_specs=[
            pl.BlockSpec((2, rb, 128), lambda i: (0, i, 0)),
            pl.BlockSpec((2, rb, 128), lambda i: (0, i, 0)),
            pl.BlockSpec((2, rb, 128), lambda i: (0, i, 0)),
            pl.BlockSpec((rb, 128), lambda i: (i, 0)),
            pl.BlockSpec((rb, 128), lambda i: (i, 0)),
            pl.BlockSpec((1, 256), lambda i: (0, 0)),
        ],
        out_specs=pl.BlockSpec((rb, 256), lambda i: (i, 0)),
        out_shape=jax.ShapeDtypeStruct((n, 256), jnp.float32),
    )(H, sa, sb, ca, cb, b2.reshape(1, -1))


def _fuse2_body(a_ref, b_ref, wt_ref, wb_ref, bias_ref, o_ref):
    y = jnp.dot(a_ref[...], wt_ref[...], preferred_element_type=jnp.float32)
    y = y + jnp.dot(b_ref[...], wb_ref[...], preferred_element_type=jnp.float32)
    o_ref[...] = jnp.maximum(y + bias_ref[...], 0.0)


def _fuse2(za, zb, W, b, rb=512):
    n = za.shape[0]
    m = W.shape[1]
    return pl.pallas_call(
        _fuse2_body,
        grid=(pl.cdiv(n, rb),),
        in_specs=[
            pl.BlockSpec((rb, 256), lambda i: (i, 0)),
            pl.BlockSpec((rb, 256), lambda i: (i, 0)),
            pl.BlockSpec((256, m), lambda i: (0, 0)),
            pl.BlockSpec((256, m), lambda i: (0, 0)),
            pl.BlockSpec((1, m), lambda i: (0, 0)),
        ],
        out_specs=pl.BlockSpec((rb, m), lambda i: (i, 0)),
        out_shape=jax.ShapeDtypeStruct((n, m), jnp.float32),
    )(za, zb, W[:256], W[256:], b.reshape(1, -1))


def _mm_body(x_ref, w_ref, o_ref):
    o_ref[...] = jnp.dot(x_ref[...], w_ref[...],
                         preferred_element_type=jnp.float32)


def _mm(x, W, rb=1024):
    n, k = x.shape
    m = W.shape[1]
    return pl.pallas_call(
        _mm_body,
        grid=(pl.cdiv(n, rb),),
        in_specs=[
            pl.BlockSpec((rb, k), lambda i: (i, 0)),
            pl.BlockSpec((k, m), lambda i: (0, 0)),
        ],
        out_specs=pl.BlockSpec((rb, m), lambda i: (i, 0)),
        out_shape=jax.ShapeDtypeStruct((n, m), jnp.float32),
    )(x, W)


def _decode_body(a_ref, bq_ref, o_ref):
    o_ref[...] = lax.dot_general(
        a_ref[...], bq_ref[...], (((1,), (1,)), ((), ())),
        preferred_element_type=jnp.float32)


def _decode(zm, zd, P, Q, rb=1024):
    a = _mm(zm, P)
    bq = _mm(zd, Q)
    n = a.shape[0]
    return pl.pallas_call(
        _decode_body,
        grid=(pl.cdiv(n, rb),),
        in_specs=[
            pl.BlockSpec((rb, a.shape[1]), lambda i: (i, 0)),
            pl.BlockSpec(bq.shape, lambda i: (0, 0)),
        ],
        out_specs=pl.BlockSpec((rb, bq.shape[0]), lambda i: (i, 0)),
        out_shape=jax.ShapeDtypeStruct((n, bq.shape[0]), jnp.float32),
    )(a, bq)


def _srcs2(src, base_part, n_rows):
    return jnp.stack([src + base_part * n_rows,
                      src + (base_part + 1) * n_rows])


def _d3(dst):
    return dst


def _gcn2(x, src, dst, cnt, p, n, n_edges):
    hp = _mm_parts(x, p["gcn1_W"], cnt, scale=True)
    s1 = _sc_scatter(hp.reshape(-1, 128), _srcs2(src, 0, n), _d3(dst), n, n_edges)
    h2p = _gcn_mid(s1.reshape(2, n, 128), hp, cnt, p["gcn1_b"], p["gcn2_W"])
    s2 = _sc_scatter(h2p.reshape(-1, 128), _srcs2(src, 0, n), _d3(dst), n, n_edges)
    return _gcn_fin(s2.reshape(2, n, 128), h2p, cnt, p["gcn2_b"])


def kernel(x_sim_m, x_sem_m, x_sim_d, x_sem_d, mm_edge, dd_edge, md_edge, dm_edge, params):
    p = params
    i32 = jnp.int32
    npad = MM_EP - mm_edge.shape[1]
    mm_src = jnp.concatenate([mm_edge[0], jnp.zeros((npad,), i32)])
    mm_dst = jnp.concatenate([mm_edge[1], jnp.full((npad,), N_M + 8, i32)])

    dstm_list = jnp.concatenate([mm_dst, md_edge[1] + N_MP])
    dstd_list = jnp.concatenate([dd_edge[1], dm_edge[1] + N_DP])
    cnt0, cnt1 = _sc_count(dstm_list, dstd_list)
    cnt_mm, cnt_md = cnt0[:N_MP], cnt0[N_MP:]
    cnt_dd, cnt_dm = cnt1[:N_DP], cnt1[N_DP:]

    m_sim = _mlp(x_sim_m, p["m_sim"], N_MP)
    d_sim = _mlp(x_sim_d, p["d_sim"], N_DP)
    zm_sim = _gcn2(m_sim, mm_src, mm_dst, cnt_mm, p, N_MP, MM_EP)
    zd_sim = _gcn2(d_sim, dd_edge[0], dd_edge[1], cnt_dd, p, N_DP, 32768)

    m_sem = _mlp(x_sem_m, p["m_sem"], N_MP)
    d_sem = _mlp(x_sem_d, p["d_sem"], N_DP)
    Wm1 = jnp.concatenate([p["rgcn1_Wroot"], p["rgcn1_Wrel"][0], p["rgcn1_Wrel"][2]], axis=1)
    Wd1 = jnp.concatenate([p["rgcn1_Wroot"], p["rgcn1_Wrel"][1], p["rgcn1_Wrel"][3]], axis=1)
    Wm2 = jnp.concatenate([p["rgcn2_Wroot"], p["rgcn2_Wrel"][0], p["rgcn2_Wrel"][2]], axis=1)
    Wd2 = jnp.concatenate([p["rgcn2_Wroot"], p["rgcn2_Wrel"][1], p["rgcn2_Wrel"][3]], axis=1)
    dummy_cnt_m = cnt_mm
    Hm = _mm_parts(m_sem, Wm1, dummy_cnt_m, scale=False)
    Hd = _mm_parts(d_sem, Wd1, cnt_dd, scale=False)

    def rel_scatters(Hm_, Hd_):
        hmf = Hm_.reshape(-1, 128)
        hdf = Hd_.reshape(-1, 128)
        s0 = _sc_scatter(hmf, _srcs2(mm_src, 2, N_MP), _d3(mm_dst), N_MP, MM_EP)
        s2 = _sc_scatter(hmf, _srcs2(md_edge[0], 4, N_MP), _d3(md_edge[1]), N_DP, 65536)
        s1 = _sc_scatter(hdf, _srcs2(dd_edge[0], 2, N_DP), _d3(dd_edge[1]), N_DP, 32768)
        s3 = _sc_scatter(hdf, _srcs2(dm_edge[0], 4, N_DP), _d3(dm_edge[1]), N_MP, 65536)
        return (s0.reshape(2, N_MP, 128), s1.reshape(2, N_DP, 128),
                s2.reshape(2, N_DP, 128), s3.reshape(2, N_MP, 128))

    s0, s1, s2, s3 = rel_scatters(Hm, Hd)
    Hm = _rgcn_mid(Hm, s0, s3, cnt_mm, cnt_dm, p["rgcn1_b"], Wm2)
    Hd = _rgcn_mid(Hd, s1, s2, cnt_dd, cnt_md, p["rgcn1_b"], Wd2)
    s0, s1, s2, s3 = rel_scatters(Hm, Hd)
    zm_sem = _rgcn_fin(Hm, s0, s3, cnt_mm, cnt_dm, p["rgcn2_b"])
    zd_sem = _rgcn_fin(Hd, s1, s2, cnt_dd, cnt_md, p["rgcn2_b"])

    zm = _fuse2(zm_sim, zm_sem, p["mf_W"], p["mf_b"])
    zd = _fuse2(zd_sim, zd_sem, p["df_W"], p["df_b"])
    logits = _decode(zm, zd, p["P"], p["Q"])
    return logits[:N_M, :N_D]
